# baseline (speedup 1.0000x reference)
import functools

import jax
import jax.numpy as jnp
from jax import lax
from jax.experimental import pallas as pl
from jax.experimental.pallas import tpu as pltpu
from jax.experimental.pallas import tpu_sc as plsc

T = 2048
D = 2048
E = 16
K = 2
FF = 1408
FFS = 2816

TILE = 384
NT = -(-(T * K) // TILE) + E
P = NT * TILE
KC = 256
NKC = D // KC
SFT = 256
NSF = FFS // SFT
META_LEN = 64
META_NU = NT
CHUNK = 256
NCH = T // CHUNK

NC, NS = 2, 16
NW = NC * NS
SC_CH = 32


def _dispatch_body(x_ref, gw_ref, pos_ref, w_ref, meta_ref):
    x = x_ref[...]
    gw = gw_ref[...]
    logits = jnp.dot(x, gw, preferred_element_type=jnp.float32)
    m = jnp.max(logits, axis=-1, keepdims=True)
    ex = jnp.exp(logits - m)
    probs = ex / jnp.sum(ex, axis=-1, keepdims=True)

    lane = lax.broadcasted_iota(jnp.int32, (T, E), 1).astype(jnp.float32)
    big = jnp.float32(1e9)
    m1 = jnp.max(probs, axis=-1, keepdims=True)
    i1 = jnp.min(jnp.where(probs == m1, lane, big), axis=-1, keepdims=True)
    probs2 = jnp.where(lane == i1, -big, probs)
    m2 = jnp.max(probs2, axis=-1, keepdims=True)
    i2 = jnp.min(jnp.where(probs2 == m2, lane, big), axis=-1, keepdims=True)

    denom = m1 + m2
    w_ref[:, 0:1] = m1 / denom
    w_ref[:, 1:2] = m2 / denom

    a0 = (lane == i1).astype(jnp.float32)
    a1 = (lane == i2).astype(jnp.float32)
    a = a0 + a1

    r = lax.broadcasted_iota(jnp.int32, (CHUNK, CHUNK), 0)
    c = lax.broadcasted_iota(jnp.int32, (CHUNK, CHUNK), 1)
    mtri = (c < r).astype(jnp.float32)
    parts = []
    prefix = jnp.zeros((1, E), jnp.float32)
    for ci in range(NCH):
        blk = a[ci * CHUNK:(ci + 1) * CHUNK, :]
        cex = jnp.dot(mtri, blk, preferred_element_type=jnp.float32)
        parts.append(cex + prefix)
        prefix = prefix + jnp.sum(blk, axis=0, keepdims=True)
    cex_all = jnp.concatenate(parts, axis=0)
    counts = prefix

    tl = jnp.float32(TILE)
    counts_i = counts.astype(jnp.int32)
    pc = (((counts_i + (TILE - 1)) // TILE) * TILE).astype(jnp.float32)
    er = lax.broadcasted_iota(jnp.int32, (E, E), 0)
    ec = lax.broadcasted_iota(jnp.int32, (E, E), 1)
    mtri_e = (er < ec).astype(jnp.float32)
    po = jnp.dot(pc, mtri_e, preferred_element_type=jnp.float32)
    pe = po + pc

    rank0 = jnp.sum(cex_all * a0, axis=-1, keepdims=True)
    rank1 = jnp.sum(cex_all * a1, axis=-1, keepdims=True)
    base0 = jnp.sum(jnp.broadcast_to(po, (T, E)) * a0, axis=-1, keepdims=True)
    base1 = jnp.sum(jnp.broadcast_to(po, (T, E)) * a1, axis=-1, keepdims=True)
    pos_ref[:, 0:1] = (base0 + rank0).astype(jnp.int32)
    pos_ref[:, 1:2] = (base1 + rank1).astype(jnp.int32)

    jt = lax.broadcasted_iota(jnp.int32, (E, NT), 1).astype(jnp.float32) * tl
    te = jnp.sum((jnp.broadcast_to(pe.reshape(E, 1), (E, NT)) <= jt)
                 .astype(jnp.float32), axis=0, keepdims=True)
    te = jnp.minimum(te, jnp.float32(E - 1))
    nu_i = jnp.sum(((counts_i + (TILE - 1)) // TILE), axis=-1, keepdims=True)
    nu = nu_i.astype(jnp.float32)
    pad = jnp.zeros((1, META_LEN - NT - 1), jnp.float32)
    meta_ref[...] = jnp.concatenate([te, nu, pad], axis=1).astype(jnp.int32)


def _dispatch(x, gate_w):
    return pl.pallas_call(
        _dispatch_body,
        out_shape=(
            jax.ShapeDtypeStruct((T, K), jnp.int32),
            jax.ShapeDtypeStruct((T, K), jnp.float32),
            jax.ShapeDtypeStruct((1, META_LEN), jnp.int32),
        ),
    )(x, gate_w)


def _sc_scatter_x(x, pos_flat):
    mesh = plsc.VectorSubcoreMesh(core_axis_name="c", subcore_axis_name="s")
    tpw = T // NW
    w = x.shape[1]

    @functools.partial(
        pl.kernel, mesh=mesh,
        out_type=jax.ShapeDtypeStruct((P, w), jnp.float32),
        scratch_types=[
            pltpu.VMEM((SC_CH,), jnp.int32),
            pltpu.VMEM((SC_CH,), jnp.int32),
            pltpu.VMEM((SC_CH, w), jnp.float32),
        ],
    )
    def k(x_hbm, pos_hbm, xs_hbm, idx0_v, idx1_v, rows_v):
        wid = lax.axis_index("s") * NC + lax.axis_index("c")
        for ci in range(tpw // SC_CH):
            base = wid * tpw + ci * SC_CH
            pltpu.sync_copy(pos_hbm.at[pl.ds(base, SC_CH)], idx0_v)
            pltpu.sync_copy(pos_hbm.at[pl.ds(T + base, SC_CH)], idx1_v)
            pltpu.sync_copy(x_hbm.at[pl.ds(base, SC_CH)], rows_v)
            pltpu.sync_copy(rows_v, xs_hbm.at[idx0_v])
            pltpu.sync_copy(rows_v, xs_hbm.at[idx1_v])

    return k(x, pos_flat)


def _sc_gather_y(y_sorted, pos_flat):
    mesh = plsc.VectorSubcoreMesh(core_axis_name="c", subcore_axis_name="s")
    rpw = (K * T) // NW
    w = y_sorted.shape[1]

    @functools.partial(
        pl.kernel, mesh=mesh,
        out_type=jax.ShapeDtypeStruct((K * T, w), jnp.float32),
        scratch_types=[
            pltpu.VMEM((SC_CH,), jnp.int32),
            pltpu.VMEM((SC_CH, w), jnp.float32),
            pltpu.SemaphoreType.DMA,
        ],
    )
    def k(y_hbm, pos_hbm, yg_hbm, idx_v, rows_v, sem):
        wid = lax.axis_index("s") * NC + lax.axis_index("c")
        for ci in range(rpw // SC_CH):
            base = wid * rpw + ci * SC_CH
            pltpu.sync_copy(pos_hbm.at[pl.ds(base, SC_CH)], idx_v)
            pltpu.async_copy(y_hbm.at[idx_v], rows_v, sem).wait()
            pltpu.sync_copy(rows_v, yg_hbm.at[pl.ds(base, SC_CH)])

    return k(y_sorted, pos_flat)


KC = 512
NKC2 = D // KC


def _gmm_body(meta_ref, x_ref, wgu_ref, wd_ref, y_ref, gu_ref):
    i = pl.program_id(0)
    j = pl.program_id(1)
    nu = meta_ref[META_NU]

    @pl.when(i < nu)
    def _():
        part = jnp.dot(x_ref[...], wgu_ref[0],
                       preferred_element_type=jnp.float32)

        @pl.when(j == 0)
        def _():
            gu_ref[...] = part

        @pl.when(j > 0)
        def _():
            gu_ref[...] += part

        @pl.when(j == NKC2 - 1)
        def _():
            g = gu_ref[:, :FF]
            u = gu_ref[:, FF:]
            h = g * jax.nn.sigmoid(g) * u
            y_ref[...] = jnp.dot(h, wd_ref[0],
                                 preferred_element_type=jnp.float32)


def _grouped_mlp(x_sorted, w_gate_up, w_down, meta):
    def ie(i, m):
        return jnp.minimum(i, m[META_NU] - 1)

    grid_spec = pltpu.PrefetchScalarGridSpec(
        num_scalar_prefetch=1,
        grid=(NT, NKC2),
        in_specs=[
            pl.BlockSpec((TILE, KC), lambda i, j, m: (ie(i, m), j)),
            pl.BlockSpec((1, KC, 2 * FF), lambda i, j, m: (m[ie(i, m)], j, 0)),
            pl.BlockSpec((1, FF, D), lambda i, j, m: (m[ie(i, m)], 0, 0)),
        ],
        out_specs=pl.BlockSpec((TILE, D), lambda i, j, m: (ie(i, m), 0)),
        scratch_shapes=[pltpu.VMEM((TILE, 2 * FF), jnp.float32)],
    )
    return pl.pallas_call(
        _gmm_body,
        grid_spec=grid_spec,
        out_shape=jax.ShapeDtypeStruct((P, D), jnp.float32),
    )(meta, x_sorted, w_gate_up, w_down)


def _shared_h_body(x_ref, sg_ref, su_ref, h_ref):
    xb = x_ref[...].astype(jnp.bfloat16)
    g = jnp.dot(xb, sg_ref[...].astype(jnp.bfloat16),
                preferred_element_type=jnp.float32)
    u = jnp.dot(xb, su_ref[...].astype(jnp.bfloat16),
                preferred_element_type=jnp.float32)
    h_ref[...] = (g * jax.nn.sigmoid(g) * u).astype(jnp.bfloat16)


def _shared_h(x, shared_gate_up, half):
    tile = T // 2
    return pl.pallas_call(
        _shared_h_body,
        grid=(NSF,),
        in_specs=[
            pl.BlockSpec((tile, D), lambda j: (half, 0)),
            pl.BlockSpec((D, SFT), lambda j: (0, j)),
            pl.BlockSpec((D, SFT), lambda j: (0, NSF + j)),
        ],
        out_specs=pl.BlockSpec((tile, SFT), lambda j: (0, j)),
        out_shape=jax.ShapeDtypeStruct((tile, FFS), jnp.bfloat16),
    )(x, shared_gate_up, shared_gate_up)


def _down_combine_body(h1_ref, h2_ref, sd_ref, ya_ref, yb_ref, w_ref, o_ref):
    i = pl.program_id(0)
    j = pl.program_id(1)

    @pl.when(j == 0)
    def _():
        o_ref[...] = (w_ref[:, 0:1] * ya_ref[...]
                      + w_ref[:, 1:2] * yb_ref[...])

    sd = sd_ref[...].astype(jnp.bfloat16)

    @pl.when(i < 2)
    def _():
        o_ref[...] += jnp.dot(h1_ref[...], sd,
                              preferred_element_type=jnp.float32)

    @pl.when(i >= 2)
    def _():
        o_ref[...] += jnp.dot(h2_ref[...], sd,
                              preferred_element_type=jnp.float32)


def _down_combine(h1, h2, shared_down, yg, topk_w):
    tile = 512
    nti = T // tile
    nh = (T // 2) // tile
    return pl.pallas_call(
        _down_combine_body,
        grid=(nti, NSF),
        in_specs=[
            pl.BlockSpec((tile, SFT),
                         lambda i, j: (jnp.minimum(i, nh - 1), j)),
            pl.BlockSpec((tile, SFT),
                         lambda i, j: (jnp.maximum(i - nh, 0), j)),
            pl.BlockSpec((SFT, D), lambda i, j: (j, 0)),
            pl.BlockSpec((tile, D), lambda i, j: (i, 0)),
            pl.BlockSpec((tile, D), lambda i, j: (i + nti, 0)),
            pl.BlockSpec((tile, K), lambda i, j: (i, 0)),
        ],
        out_specs=pl.BlockSpec((tile, D), lambda i, j: (i, 0)),
        out_shape=jax.ShapeDtypeStruct((T, D), jnp.float32),
    )(h1, h2, shared_down, yg, yg, topk_w)


def kernel(hidden_states, gate_w, w_gate_up, w_down, shared_gate_up,
           shared_down):
    x = hidden_states
    pos, topk_w, meta2d = _dispatch(x, gate_w)
    pos_flat = pos.T.reshape(K * T)
    meta = meta2d.reshape(META_LEN)

    x_sorted = _sc_scatter_x(x, pos_flat)
    h1 = _shared_h(x, shared_gate_up, 0)
    y_sorted = _grouped_mlp(x_sorted, w_gate_up, w_down, meta)
    h2 = _shared_h(x, shared_gate_up, 1)
    yg = _sc_gather_y(y_sorted, pos_flat)
    return _down_combine(h1, h2, shared_down, yg, topk_w)


# speedup vs baseline: 1.0197x; 1.0197x over previous
import functools

import jax
import jax.numpy as jnp
from jax import lax
from jax.experimental import pallas as pl
from jax.experimental.pallas import tpu as pltpu
from jax.experimental.pallas import tpu_sc as plsc

T = 2048
D = 2048
E = 16
K = 2
FF = 1408
FFS = 2816

TILE = 384
NT = -(-(T * K) // TILE) + E
P = NT * TILE
KC = 256
NKC = D // KC
SFT = 256
NSF = FFS // SFT
META_LEN = 64
META_NU = NT
CHUNK = 256
NCH = T // CHUNK

NC, NS = 2, 16
NW = NC * NS
SC_CH = 32


def _dispatch_body(x_ref, gw_ref, pos_ref, w_ref, meta_ref):
    x = x_ref[...]
    gw = gw_ref[...]
    logits = jnp.dot(x, gw, preferred_element_type=jnp.float32)
    m = jnp.max(logits, axis=-1, keepdims=True)
    ex = jnp.exp(logits - m)
    probs = ex / jnp.sum(ex, axis=-1, keepdims=True)

    lane = lax.broadcasted_iota(jnp.int32, (T, E), 1).astype(jnp.float32)
    big = jnp.float32(1e9)
    m1 = jnp.max(probs, axis=-1, keepdims=True)
    i1 = jnp.min(jnp.where(probs == m1, lane, big), axis=-1, keepdims=True)
    probs2 = jnp.where(lane == i1, -big, probs)
    m2 = jnp.max(probs2, axis=-1, keepdims=True)
    i2 = jnp.min(jnp.where(probs2 == m2, lane, big), axis=-1, keepdims=True)

    denom = m1 + m2
    w_ref[:, 0:1] = m1 / denom
    w_ref[:, 1:2] = m2 / denom

    a0 = (lane == i1).astype(jnp.float32)
    a1 = (lane == i2).astype(jnp.float32)
    a = a0 + a1

    r = lax.broadcasted_iota(jnp.int32, (CHUNK, CHUNK), 0)
    c = lax.broadcasted_iota(jnp.int32, (CHUNK, CHUNK), 1)
    mtri = (c < r).astype(jnp.float32)
    parts = []
    prefix = jnp.zeros((1, E), jnp.float32)
    for ci in range(NCH):
        blk = a[ci * CHUNK:(ci + 1) * CHUNK, :]
        cex = jnp.dot(mtri, blk, preferred_element_type=jnp.float32)
        parts.append(cex + prefix)
        prefix = prefix + jnp.sum(blk, axis=0, keepdims=True)
    cex_all = jnp.concatenate(parts, axis=0)
    counts = prefix

    tl = jnp.float32(TILE)
    counts_i = counts.astype(jnp.int32)
    pc = (((counts_i + (TILE - 1)) // TILE) * TILE).astype(jnp.float32)
    er = lax.broadcasted_iota(jnp.int32, (E, E), 0)
    ec = lax.broadcasted_iota(jnp.int32, (E, E), 1)
    mtri_e = (er < ec).astype(jnp.float32)
    po = jnp.dot(pc, mtri_e, preferred_element_type=jnp.float32)
    pe = po + pc

    rank0 = jnp.sum(cex_all * a0, axis=-1, keepdims=True)
    rank1 = jnp.sum(cex_all * a1, axis=-1, keepdims=True)
    base0 = jnp.sum(jnp.broadcast_to(po, (T, E)) * a0, axis=-1, keepdims=True)
    base1 = jnp.sum(jnp.broadcast_to(po, (T, E)) * a1, axis=-1, keepdims=True)
    pos_ref[:, 0:1] = (base0 + rank0).astype(jnp.int32)
    pos_ref[:, 1:2] = (base1 + rank1).astype(jnp.int32)

    jt = lax.broadcasted_iota(jnp.int32, (E, NT), 1).astype(jnp.float32) * tl
    te = jnp.sum((jnp.broadcast_to(pe.reshape(E, 1), (E, NT)) <= jt)
                 .astype(jnp.float32), axis=0, keepdims=True)
    te = jnp.minimum(te, jnp.float32(E - 1))
    nu_i = jnp.sum(((counts_i + (TILE - 1)) // TILE), axis=-1, keepdims=True)
    nu = nu_i.astype(jnp.float32)
    pad = jnp.zeros((1, META_LEN - NT - 1), jnp.float32)
    meta_ref[...] = jnp.concatenate([te, nu, pad], axis=1).astype(jnp.int32)


def _dispatch(x, gate_w):
    return pl.pallas_call(
        _dispatch_body,
        out_shape=(
            jax.ShapeDtypeStruct((T, K), jnp.int32),
            jax.ShapeDtypeStruct((T, K), jnp.float32),
            jax.ShapeDtypeStruct((1, META_LEN), jnp.int32),
        ),
    )(x, gate_w)


def _sc_scatter_x(x, pos_flat):
    mesh = plsc.VectorSubcoreMesh(core_axis_name="c", subcore_axis_name="s")
    tpw = T // NW
    w = x.shape[1]

    @functools.partial(
        pl.kernel, mesh=mesh,
        out_type=jax.ShapeDtypeStruct((P, w), jnp.float32),
        scratch_types=[
            pltpu.VMEM((SC_CH,), jnp.int32),
            pltpu.VMEM((SC_CH,), jnp.int32),
            pltpu.VMEM((SC_CH, w), jnp.float32),
        ],
    )
    def k(x_hbm, pos_hbm, xs_hbm, idx0_v, idx1_v, rows_v):
        wid = lax.axis_index("s") * NC + lax.axis_index("c")
        for ci in range(tpw // SC_CH):
            base = wid * tpw + ci * SC_CH
            pltpu.sync_copy(pos_hbm.at[pl.ds(base, SC_CH)], idx0_v)
            pltpu.sync_copy(pos_hbm.at[pl.ds(T + base, SC_CH)], idx1_v)
            pltpu.sync_copy(x_hbm.at[pl.ds(base, SC_CH)], rows_v)
            pltpu.sync_copy(rows_v, xs_hbm.at[idx0_v])
            pltpu.sync_copy(rows_v, xs_hbm.at[idx1_v])

    return k(x, pos_flat)


def _sc_gather_y(y_sorted, pos_flat):
    mesh = plsc.VectorSubcoreMesh(core_axis_name="c", subcore_axis_name="s")
    rpw = (K * T) // NW
    w = y_sorted.shape[1]

    @functools.partial(
        pl.kernel, mesh=mesh,
        out_type=jax.ShapeDtypeStruct((K * T, w), jnp.float32),
        scratch_types=[
            pltpu.VMEM((SC_CH,), jnp.int32),
            pltpu.VMEM((SC_CH, w), jnp.float32),
            pltpu.SemaphoreType.DMA,
        ],
    )
    def k(y_hbm, pos_hbm, yg_hbm, idx_v, rows_v, sem):
        wid = lax.axis_index("s") * NC + lax.axis_index("c")
        for ci in range(rpw // SC_CH):
            base = wid * rpw + ci * SC_CH
            pltpu.sync_copy(pos_hbm.at[pl.ds(base, SC_CH)], idx_v)
            pltpu.async_copy(y_hbm.at[idx_v], rows_v, sem).wait()
            pltpu.sync_copy(rows_v, yg_hbm.at[pl.ds(base, SC_CH)])

    return k(y_sorted, pos_flat)


KC = 512
NKC2 = D // KC


def _gmm_body(meta_ref, x_ref, wgu_ref, wd_ref, y_ref, gu_ref):
    i = pl.program_id(0)
    j = pl.program_id(1)
    nu = meta_ref[META_NU]

    @pl.when(i < nu)
    def _():
        part = jnp.dot(x_ref[...], wgu_ref[0],
                       preferred_element_type=jnp.float32)

        @pl.when(j == 0)
        def _():
            gu_ref[...] = part

        @pl.when(j > 0)
        def _():
            gu_ref[...] += part

        @pl.when(j == NKC2 - 1)
        def _():
            g = gu_ref[:, :FF]
            u = gu_ref[:, FF:]
            h = g * jax.nn.sigmoid(g) * u
            y_ref[...] = jnp.dot(h, wd_ref[0],
                                 preferred_element_type=jnp.float32)


def _grouped_mlp(x_sorted, w_gate_up, w_down, meta):
    def ie(i, m):
        return jnp.minimum(i, m[META_NU] - 1)

    grid_spec = pltpu.PrefetchScalarGridSpec(
        num_scalar_prefetch=1,
        grid=(NT, NKC2),
        in_specs=[
            pl.BlockSpec((TILE, KC), lambda i, j, m: (ie(i, m), j)),
            pl.BlockSpec((1, KC, 2 * FF), lambda i, j, m: (m[ie(i, m)], j, 0)),
            pl.BlockSpec((1, FF, D), lambda i, j, m: (m[ie(i, m)], 0, 0)),
        ],
        out_specs=pl.BlockSpec((TILE, D), lambda i, j, m: (ie(i, m), 0)),
        scratch_shapes=[pltpu.VMEM((TILE, 2 * FF), jnp.float32)],
    )
    return pl.pallas_call(
        _gmm_body,
        grid_spec=grid_spec,
        out_shape=jax.ShapeDtypeStruct((P, D), jnp.float32),
    )(meta, x_sorted, w_gate_up, w_down)


def _shared_h_body(x_ref, sg_ref, su_ref, h_ref):
    xb = x_ref[...].astype(jnp.bfloat16)
    g = jnp.dot(xb, sg_ref[...].astype(jnp.bfloat16),
                preferred_element_type=jnp.float32)
    u = jnp.dot(xb, su_ref[...].astype(jnp.bfloat16),
                preferred_element_type=jnp.float32)
    h_ref[...] = (g * jax.nn.sigmoid(g) * u).astype(jnp.bfloat16)


def _shared_h(x, shared_gate_up, half):
    tile = T // 2
    return pl.pallas_call(
        _shared_h_body,
        grid=(NSF,),
        in_specs=[
            pl.BlockSpec((tile, D), lambda j: (half, 0)),
            pl.BlockSpec((D, SFT), lambda j: (0, j)),
            pl.BlockSpec((D, SFT), lambda j: (0, NSF + j)),
        ],
        out_specs=pl.BlockSpec((tile, SFT), lambda j: (0, j)),
        out_shape=jax.ShapeDtypeStruct((tile, FFS), jnp.bfloat16),
    )(x, shared_gate_up, shared_gate_up)


def _down_combine_body(h1_ref, h2_ref, sd_ref, ya_ref, yb_ref, w_ref, o_ref):
    i = pl.program_id(0)
    j = pl.program_id(1)

    @pl.when(j == 0)
    def _():
        o_ref[...] = (w_ref[:, 0:1] * ya_ref[...]
                      + w_ref[:, 1:2] * yb_ref[...])

    sd = sd_ref[...].astype(jnp.bfloat16)

    @pl.when(i < 2)
    def _():
        o_ref[...] += jnp.dot(h1_ref[...], sd,
                              preferred_element_type=jnp.float32)

    @pl.when(i >= 2)
    def _():
        o_ref[...] += jnp.dot(h2_ref[...], sd,
                              preferred_element_type=jnp.float32)


def _down_combine(h1, h2, shared_down, yg, topk_w):
    tile = 512
    nti = T // tile
    nh = (T // 2) // tile
    return pl.pallas_call(
        _down_combine_body,
        grid=(nti, NSF),
        in_specs=[
            pl.BlockSpec((tile, SFT),
                         lambda i, j: (jnp.minimum(i, nh - 1), j)),
            pl.BlockSpec((tile, SFT),
                         lambda i, j: (jnp.maximum(i - nh, 0), j)),
            pl.BlockSpec((SFT, D), lambda i, j: (j, 0)),
            pl.BlockSpec((tile, D), lambda i, j: (i, 0)),
            pl.BlockSpec((tile, D), lambda i, j: (i + nti, 0)),
            pl.BlockSpec((tile, K), lambda i, j: (i, 0)),
        ],
        out_specs=pl.BlockSpec((tile, D), lambda i, j: (i, 0)),
        out_shape=jax.ShapeDtypeStruct((T, D), jnp.float32),
    )(h1, h2, shared_down, yg, yg, topk_w)


def kernel(hidden_states, gate_w, w_gate_up, w_down, shared_gate_up,
           shared_down):
    x = hidden_states
    pos, topk_w, meta2d = _dispatch(x, gate_w)
    pos_flat = pos.T.reshape(K * T)
    meta = meta2d.reshape(META_LEN)

    x_sorted = _sc_scatter_x(x, pos_flat)
    h1 = _shared_h(x, shared_gate_up, 0)
    meta_d = meta + (h1[0, 0] * 0).astype(jnp.int32)
    y_sorted = _grouped_mlp(x_sorted, w_gate_up, w_down, meta_d)
    h2 = _shared_h(x, shared_gate_up, 1)
    yg = _sc_gather_y(y_sorted, pos_flat)
    return _down_combine(h1, h2, shared_down, yg, topk_w)
